# Initial kernel scaffold; baseline (speedup 1.0000x reference)
import jax
import jax.numpy as jnp
from jax.experimental import pallas as pl


def kernel(edge_index_dd, edge_index_rev, drug_idx, disease_idx, drug_table, disease_table, sage_wl, sage_bl, sage_wr, attn_in_w, attn_in_b, attn_out_w, attn_out_b, mlp_w1, mlp_b1, mlp_w2, mlp_b2, mlp_w3, mlp_b3):
    raise NotImplementedError("write your pallas kernel here")


# speedup vs baseline: 3.2362x; 3.2362x over previous
import functools

import jax
import jax.numpy as jnp
from jax import lax
from jax.experimental import pallas as pl
from jax.experimental.pallas import tpu as pltpu
from jax.experimental.pallas import tpu_sc as plsc

_N = 10000
_NPAD = 10240
_E = 320000
_H = 128
_B = 4096
_NC = 2
_NS = 16
_CH = 80
_RPT = _NPAD // _NS

_f32 = jnp.float32


def _mesh():
    return plsc.VectorSubcoreMesh(core_axis_name="c", subcore_axis_name="s")


def _segsum_body(table, src, dst, zrows, out, acc, idx_s, idx_d, rows, sem):
    c = lax.axis_index("c")
    s = lax.axis_index("s")
    pltpu.sync_copy(zrows, acc.at[pl.ds(s * _RPT, _RPT)])
    plsc.subcore_barrier()
    ept = _E // (_NC * _NS)
    base = c * (_E // _NC) + s * ept

    def step(i, carry):
        off = base + i * _CH
        pltpu.sync_copy(src.at[pl.ds(off, _CH)], idx_s)
        pltpu.sync_copy(dst.at[pl.ds(off, _CH)], idx_d)
        pltpu.async_copy(table.at[idx_s], rows, sem).wait()
        pltpu.sync_copy(rows, acc.at[idx_d], add=True)
        return carry

    lax.fori_loop(0, ept // _CH, step, 0)
    plsc.subcore_barrier()
    pltpu.sync_copy(acc.at[pl.ds(s * _RPT, _RPT)],
                    out.at[pl.ds(c * _NPAD + s * _RPT, _RPT)])


_segsum = functools.partial(
    pl.kernel,
    _segsum_body,
    out_type=jax.ShapeDtypeStruct((_NC * _NPAD, _H), _f32),
    mesh=_mesh(),
    scratch_types=[
        pltpu.VMEM_SHARED((_NPAD, _H), _f32),
        pltpu.VMEM((_CH,), jnp.int32),
        pltpu.VMEM((_CH,), jnp.int32),
        pltpu.VMEM((_CH, _H), _f32),
        pltpu.SemaphoreType.DMA,
    ],
)()


def _counts_body(dsts, ones_h, zc, out, acc, idx_d, ones_v, sem):
    c = lax.axis_index("c")
    s = lax.axis_index("s")
    pltpu.sync_copy(zc, acc.at[pl.ds(s * _RPT, _RPT)])
    pltpu.sync_copy(ones_h, ones_v)
    plsc.subcore_barrier()
    ept = _E // _NS
    base = c * _E + s * ept

    def step(i, carry):
        off = base + i * _CH
        pltpu.sync_copy(dsts.at[pl.ds(off, _CH)], idx_d)
        pltpu.sync_copy(ones_v, acc.at[idx_d], add=True)
        return carry

    lax.fori_loop(0, ept // _CH, step, 0)
    plsc.subcore_barrier()
    pltpu.sync_copy(acc.at[pl.ds(s * _RPT, _RPT)],
                    out.at[pl.ds(c * _NPAD + s * _RPT, _RPT)])


_counts = functools.partial(
    pl.kernel,
    _counts_body,
    out_type=jax.ShapeDtypeStruct((_NC * _NPAD,), _f32),
    mesh=_mesh(),
    scratch_types=[
        pltpu.VMEM_SHARED((_NPAD,), _f32),
        pltpu.VMEM((_CH,), jnp.int32),
        pltpu.VMEM((_CH,), _f32),
        pltpu.SemaphoreType.DMA,
    ],
)()


def _gather_body(sum_d, xd0, rc_d, sum_s, xs1, rc_s, di, dip, si, sip,
                 gd0, gd1, gdx, gcd, gs0, gs1, gsx, gcs,
                 idx, rows, sem):
    c = lax.axis_index("c")
    s = lax.axis_index("s")
    wid = s * _NC + c
    rpt = _B // (_NC * _NS)
    base = wid * rpt

    pltpu.sync_copy(di.at[pl.ds(base, rpt)], idx)
    pltpu.async_copy(sum_d.at[idx], rows, sem).wait()
    pltpu.sync_copy(rows, gd0.at[pl.ds(base, rpt)])
    pltpu.async_copy(xd0.at[idx], rows, sem).wait()
    pltpu.sync_copy(rows, gdx.at[pl.ds(base, rpt)])
    pltpu.async_copy(rc_d.at[idx], rows, sem).wait()
    pltpu.sync_copy(rows, gcd.at[pl.ds(base, rpt)])
    pltpu.sync_copy(dip.at[pl.ds(base, rpt)], idx)
    pltpu.async_copy(sum_d.at[idx], rows, sem).wait()
    pltpu.sync_copy(rows, gd1.at[pl.ds(base, rpt)])
    pltpu.sync_copy(si.at[pl.ds(base, rpt)], idx)
    pltpu.async_copy(sum_s.at[idx], rows, sem).wait()
    pltpu.sync_copy(rows, gs0.at[pl.ds(base, rpt)])
    pltpu.async_copy(xs1.at[idx], rows, sem).wait()
    pltpu.sync_copy(rows, gsx.at[pl.ds(base, rpt)])
    pltpu.async_copy(rc_s.at[idx], rows, sem).wait()
    pltpu.sync_copy(rows, gcs.at[pl.ds(base, rpt)])
    pltpu.sync_copy(sip.at[pl.ds(base, rpt)], idx)
    pltpu.async_copy(sum_s.at[idx], rows, sem).wait()
    pltpu.sync_copy(rows, gs1.at[pl.ds(base, rpt)])


_gather = functools.partial(
    pl.kernel,
    _gather_body,
    out_type=[jax.ShapeDtypeStruct((_B, _H), _f32)] * 8,
    mesh=_mesh(),
    scratch_types=[
        pltpu.VMEM((_B // (_NC * _NS),), jnp.int32),
        pltpu.VMEM((_B // (_NC * _NS), _H), _f32),
        pltpu.SemaphoreType.DMA,
    ],
)()


def _d0_body(sa_dis, sb_dis, cd, xt_dis, sa_drug, sb_drug, cg, xt_drug,
             w_dis, b_dis, w_drug, b_drug,
             xs0, xs1, xd0, xd1, rc_dis, rc_drug):
    dot = functools.partial(jnp.dot, preferred_element_type=_f32)
    rcd = 1.0 / jnp.maximum(cd[:, 0:1], 1.0)
    rcg = 1.0 / jnp.maximum(cg[:, 0:1], 1.0)
    rc_dis[...] = jnp.broadcast_to(rcd, rc_dis.shape)
    rc_drug[...] = jnp.broadcast_to(rcg, rc_drug.shape)
    agg_dis = (sa_dis[...] + sb_dis[...]) * rcd
    agg_drug = (sa_drug[...] + sb_drug[...]) * rcg
    xs0[...] = jnp.maximum(
        dot(agg_dis, w_dis[0]) + b_dis[0] + dot(xt_dis[...], w_dis[1]), 0.0)
    xs1[...] = jnp.maximum(
        dot(agg_dis, w_dis[2]) + b_dis[1] + dot(xt_dis[...], w_dis[3]), 0.0)
    xd0[...] = jnp.maximum(
        dot(agg_drug, w_drug[0]) + b_drug[0] + dot(xt_drug[...], w_drug[1]), 0.0)
    xd1[...] = jnp.maximum(
        dot(agg_drug, w_drug[2]) + b_drug[1] + dot(xt_drug[...], w_drug[3]), 0.0)


def _run_d0(sa_dis, sb_dis, cd, xt_dis, sa_drug, sb_drug, cg, xt_drug,
            w_dis, b_dis, w_drug, b_drug):
    r = 1000
    grid = (_N // r,)
    row = pl.BlockSpec((r, _H), lambda i: (i, 0))
    cblk = pl.BlockSpec((r, 1), lambda i: (i, 0))
    wblk = pl.BlockSpec((4, _H, _H), lambda i: (0, 0, 0))
    bblk = pl.BlockSpec((2, _H), lambda i: (0, 0))
    return pl.pallas_call(
        _d0_body,
        grid=grid,
        in_specs=[row, row, cblk, row, row, row, cblk, row,
                  wblk, bblk, wblk, bblk],
        out_specs=[row, row, row, row, row, row],
        out_shape=[jax.ShapeDtypeStruct((_N, _H), _f32)] * 6,
    )(sa_dis, sb_dis, cd, xt_dis, sa_drug, sb_drug, cg, xt_drug,
      w_dis, b_dis, w_drug, b_drug)


def _d1_body(gd0, gd1, gdx, gcd, gs0, gs1, gsx, gcs,
             w_d, b_d, w_s, b_s, wv_t, bv, wo_t, bo,
             w1, b1, w2, b2, w3r, out):
    dot = functools.partial(jnp.dot, preferred_element_type=_f32)
    demb = jnp.maximum(
        dot((gd0[...] + gd1[...]) * gcd[...], w_d[0])
        + b_d[0] + dot(gdx[...], w_d[1]), 0.0)
    semb = jnp.maximum(
        dot((gs0[...] + gs1[...]) * gcs[...], w_s[0])
        + b_s[0] + dot(gsx[...], w_s[1]), 0.0)
    datt = dot(dot(semb, wv_t[0]) + bv[0], wo_t[0]) + bo[0]
    satt = dot(dot(demb, wv_t[1]) + bv[1], wo_t[1]) + bo[1]
    li = jnp.concatenate([demb, semb, datt, satt], axis=-1)
    h = jnp.maximum(dot(li, w1[...]) + b1[0], 0.0)
    h = jnp.maximum(dot(h, w2[...]) + b2[0], 0.0)
    o = jnp.sum(h * w3r[0:1, :], axis=-1) + w3r[1, 0]
    out[...] = o.reshape(out.shape)


def _run_d1(gd0, gd1, gdx, gcd, gs0, gs1, gsx, gcs,
            w_d, b_d, w_s, b_s, wv_t, bv, wo_t, bo, w1, b1, w2, b2, w3r):
    r = 1024
    grid = (_B // r,)
    row = pl.BlockSpec((r, _H), lambda i: (i, 0))
    w2blk = pl.BlockSpec((2, _H, _H), lambda i: (0, 0, 0))
    bblk = pl.BlockSpec((2, _H), lambda i: (0, 0))
    full = lambda a: pl.BlockSpec(a.shape, lambda i: tuple(0 for _ in a.shape))
    return pl.pallas_call(
        _d1_body,
        grid=grid,
        in_specs=[row, row, row, row, row, row, row, row,
                  w2blk, bblk, w2blk, bblk, w2blk, bblk, w2blk, bblk,
                  full(w1), full(b1), full(w2), full(b2), full(w3r)],
        out_specs=[pl.BlockSpec((r // _H, _H), lambda i: (i, 0))],
        out_shape=[jax.ShapeDtypeStruct((_B // _H, _H), _f32)],
    )(gd0, gd1, gdx, gcd, gs0, gs1, gsx, gcs,
      w_d, b_d, w_s, b_s, wv_t, bv, wo_t, bo, w1, b1, w2, b2, w3r)[0]


def kernel(edge_index_dd, edge_index_rev, drug_idx, disease_idx, drug_table,
           disease_table, sage_wl, sage_bl, sage_wr, attn_in_w, attn_in_b,
           attn_out_w, attn_out_b, mlp_w1, mlp_b1, mlp_w2, mlp_b2, mlp_w3,
           mlp_b3):
    i32 = jnp.int32
    src_dd = edge_index_dd[0].astype(i32)
    dst_dd = edge_index_dd[1].astype(i32)
    src_rev = edge_index_rev[0].astype(i32)
    dst_rev = edge_index_rev[1].astype(i32)
    di = drug_idx.astype(i32)
    si = disease_idx.astype(i32)
    dip = di + _NPAD
    sip = si + _NPAD

    zrows = jnp.zeros((_RPT, _H), _f32)
    zc = jnp.zeros((_RPT,), _f32)
    ones_h = jnp.ones((_CH,), _f32)

    cnts = _counts(jnp.concatenate([dst_dd, dst_rev]), ones_h, zc)
    cnt_dis = cnts[:_N].reshape(_N, 1)
    cnt_drug = cnts[_NPAD:_NPAD + _N].reshape(_N, 1)

    sum_dis0 = _segsum(drug_table, src_dd, dst_dd, zrows)
    sum_drug0 = _segsum(disease_table, src_rev, dst_rev, zrows)

    h = _H
    w_dis = jnp.stack([sage_wl[0, 0, 0], sage_wr[0, 0, 0],
                       sage_wl[1, 0, 0], sage_wr[1, 0, 0]])
    b_dis = jnp.stack([sage_bl[0, 0, 0], sage_bl[1, 0, 0]])
    w_drug = jnp.stack([sage_wl[0, 0, 1], sage_wr[0, 0, 1],
                        sage_wl[1, 0, 1], sage_wr[1, 0, 1]])
    b_drug = jnp.stack([sage_bl[0, 0, 1], sage_bl[1, 0, 1]])

    xs0, xs1, xd0, xd1, rc_dis, rc_drug = _run_d0(
        sum_dis0[:_N], sum_dis0[_NPAD:_NPAD + _N], cnt_dis, disease_table,
        sum_drug0[:_N], sum_drug0[_NPAD:_NPAD + _N], cnt_drug, drug_table,
        w_dis, b_dis, w_drug, b_drug)

    sum_drug1 = _segsum(xs0, src_rev, dst_rev, zrows)
    sum_dis1 = _segsum(xd1, src_dd, dst_dd, zrows)

    gd0, gd1, gdx, gcd, gs0, gs1, gsx, gcs = _gather(
        sum_drug1, xd0, rc_drug, sum_dis1, xs1, rc_dis, di, dip, si, sip)

    w_d = jnp.stack([sage_wl[0, 1, 1], sage_wr[0, 1, 1]])
    b_d = jnp.stack([sage_bl[0, 1, 1], sage_bl[0, 1, 1]])
    w_s = jnp.stack([sage_wl[1, 1, 0], sage_wr[1, 1, 0]])
    b_s = jnp.stack([sage_bl[1, 1, 0], sage_bl[1, 1, 0]])
    wv_t = jnp.stack([attn_in_w[0, 2 * h:3 * h].T, attn_in_w[1, 2 * h:3 * h].T])
    bv = jnp.stack([attn_in_b[0, 2 * h:3 * h], attn_in_b[1, 2 * h:3 * h]])
    wo_t = jnp.stack([attn_out_w[0].T, attn_out_w[1].T])
    bo = jnp.stack([attn_out_b[0], attn_out_b[1]])
    b1 = mlp_b1.reshape(1, -1)
    b2 = mlp_b2.reshape(1, -1)
    w3r = jnp.concatenate(
        [mlp_w3[:, 0:1].T, jnp.full((1, _H), mlp_b3[0], _f32)], axis=0)

    out = _run_d1(gd0, gd1, gdx, gcd, gs0, gs1, gsx, gcs,
                  w_d, b_d, w_s, b_s, wv_t, bv, wo_t, bo,
                  mlp_w1, b1, mlp_w2, b2, w3r)
    return out.reshape(_B)


# speedup vs baseline: 7.8639x; 2.4300x over previous
import functools

import jax
import jax.numpy as jnp
from jax import lax
from jax.experimental import pallas as pl
from jax.experimental.pallas import tpu as pltpu
from jax.experimental.pallas import tpu_sc as plsc

_N = 10000
_NPAD = 10240
_E = 320000
_H = 128
_B = 4096
_NC = 2
_NS = 16
_CH = 80
_RPT = _NPAD // _NS

_f32 = jnp.float32


def _mesh():
    return plsc.VectorSubcoreMesh(core_axis_name="c", subcore_axis_name="s")


def _make_segsum_body(with_counts):
    def body(table, src, dst, zrows, *rest):
        if with_counts:
            (ones_hb, zc, out, cout, acc, cacc, rows0, rows1, dch0, dch1,
             src_all, ones_v, semg0, semg1, semd0, semd1) = rest
        else:
            (out, acc, rows0, rows1, dch0, dch1,
             src_all, semg0, semg1, semd0, semd1) = rest
        c = lax.axis_index("c")
        s = lax.axis_index("s")
        pltpu.sync_copy(zrows, acc.at[pl.ds(s * _RPT, _RPT)])
        if with_counts:
            pltpu.sync_copy(zc, cacc.at[pl.ds(s * _RPT, _RPT)])
            pltpu.sync_copy(ones_hb, ones_v)
        ept = _E // (_NC * _NS)
        base = c * (_E // _NC) + s * ept
        pltpu.sync_copy(src.at[pl.ds(base, ept)], src_all)
        plsc.subcore_barrier()

        def fire(i, dch, rows, semd, semg):
            pltpu.async_copy(dst.at[pl.ds(base + i * _CH, _CH)], dch, semd)
            pltpu.async_copy(table.at[src_all.at[pl.ds(i * _CH, _CH)]],
                             rows, semg)

        def drain(dch, rows, semd, semg):
            pltpu.make_async_copy(dst.at[pl.ds(base, _CH)], dch, semd).wait()
            pltpu.make_async_copy(table.at[src_all.at[pl.ds(0, _CH)]],
                                  rows, semg).wait()

        def scat(dch, rows):
            pltpu.sync_copy(rows, acc.at[dch], add=True)
            if with_counts:
                pltpu.sync_copy(ones_v, cacc.at[dch], add=True)

        nch = ept // _CH
        fire(0, dch0, rows0, semd0, semg0)

        def step(k, carry):
            i0 = k * 2
            fire(i0 + 1, dch1, rows1, semd1, semg1)
            drain(dch0, rows0, semd0, semg0)
            scat(dch0, rows0)
            fire(i0 + 2, dch0, rows0, semd0, semg0)
            drain(dch1, rows1, semd1, semg1)
            scat(dch1, rows1)
            return carry

        lax.fori_loop(0, nch // 2, step, 0)
        drain(dch0, rows0, semd0, semg0)
        scat(dch0, rows0)
        plsc.subcore_barrier()
        pltpu.sync_copy(acc.at[pl.ds(s * _RPT, _RPT)],
                        out.at[pl.ds(c * _NPAD + s * _RPT, _RPT)])
        if with_counts:
            pltpu.sync_copy(cacc.at[pl.ds(s * _RPT, _RPT)],
                            cout.at[pl.ds(c * _NPAD + s * _RPT, _RPT)])

    return body


_SEG_SCRATCH = [
    pltpu.VMEM((_CH, _H), _f32),
    pltpu.VMEM((_CH, _H), _f32),
    pltpu.VMEM((_CH,), jnp.int32),
    pltpu.VMEM((_CH,), jnp.int32),
    pltpu.VMEM((_E // (_NC * _NS),), jnp.int32),
]
_SEG_SEMS = [pltpu.SemaphoreType.DMA] * 4

_segsum = functools.partial(
    pl.kernel,
    _make_segsum_body(False),
    out_type=jax.ShapeDtypeStruct((_NC * _NPAD, _H), _f32),
    mesh=_mesh(),
    scratch_types=[pltpu.VMEM_SHARED((_NPAD, _H), _f32)]
    + _SEG_SCRATCH + _SEG_SEMS,
)()

_segsum_cnt = functools.partial(
    pl.kernel,
    _make_segsum_body(True),
    out_type=[jax.ShapeDtypeStruct((_NC * _NPAD, _H), _f32),
              jax.ShapeDtypeStruct((_NC * _NPAD,), _f32)],
    mesh=_mesh(),
    scratch_types=[pltpu.VMEM_SHARED((_NPAD, _H), _f32),
                   pltpu.VMEM_SHARED((_NPAD,), _f32)]
    + _SEG_SCRATCH + [pltpu.VMEM((_CH,), _f32)] + _SEG_SEMS,
)()


def _gather_body(sum_d, xd0, rc_d, sum_s, xs1, rc_s, di, dip, si, sip,
                 gd0, gd1, gdx, gcd, gs0, gs1, gsx, gcs,
                 idx, rows, sem):
    c = lax.axis_index("c")
    s = lax.axis_index("s")
    wid = s * _NC + c
    rpt = _B // (_NC * _NS)
    base = wid * rpt

    pltpu.sync_copy(di.at[pl.ds(base, rpt)], idx)
    pltpu.async_copy(sum_d.at[idx], rows, sem).wait()
    pltpu.sync_copy(rows, gd0.at[pl.ds(base, rpt)])
    pltpu.async_copy(xd0.at[idx], rows, sem).wait()
    pltpu.sync_copy(rows, gdx.at[pl.ds(base, rpt)])
    pltpu.async_copy(rc_d.at[idx], rows, sem).wait()
    pltpu.sync_copy(rows, gcd.at[pl.ds(base, rpt)])
    pltpu.sync_copy(dip.at[pl.ds(base, rpt)], idx)
    pltpu.async_copy(sum_d.at[idx], rows, sem).wait()
    pltpu.sync_copy(rows, gd1.at[pl.ds(base, rpt)])
    pltpu.sync_copy(si.at[pl.ds(base, rpt)], idx)
    pltpu.async_copy(sum_s.at[idx], rows, sem).wait()
    pltpu.sync_copy(rows, gs0.at[pl.ds(base, rpt)])
    pltpu.async_copy(xs1.at[idx], rows, sem).wait()
    pltpu.sync_copy(rows, gsx.at[pl.ds(base, rpt)])
    pltpu.async_copy(rc_s.at[idx], rows, sem).wait()
    pltpu.sync_copy(rows, gcs.at[pl.ds(base, rpt)])
    pltpu.sync_copy(sip.at[pl.ds(base, rpt)], idx)
    pltpu.async_copy(sum_s.at[idx], rows, sem).wait()
    pltpu.sync_copy(rows, gs1.at[pl.ds(base, rpt)])


_gather = functools.partial(
    pl.kernel,
    _gather_body,
    out_type=[jax.ShapeDtypeStruct((_B, _H), _f32)] * 8,
    mesh=_mesh(),
    scratch_types=[
        pltpu.VMEM((_B // (_NC * _NS),), jnp.int32),
        pltpu.VMEM((_B // (_NC * _NS), _H), _f32),
        pltpu.SemaphoreType.DMA,
    ],
)()


def _d0_body(sa_dis, sb_dis, cda, cdb, xt_dis, sa_drug, sb_drug, cga, cgb,
             xt_drug, w_dis, b_dis, w_drug, b_drug,
             xs0, xs1, xd0, xd1, rc_dis, rc_drug):
    dot = functools.partial(jnp.dot, preferred_element_type=_f32)
    rcd = 1.0 / jnp.maximum(cda[:, 0:1] + cdb[:, 0:1], 1.0)
    rcg = 1.0 / jnp.maximum(cga[:, 0:1] + cgb[:, 0:1], 1.0)
    rc_dis[...] = jnp.broadcast_to(rcd, rc_dis.shape)
    rc_drug[...] = jnp.broadcast_to(rcg, rc_drug.shape)
    agg_dis = (sa_dis[...] + sb_dis[...]) * rcd
    agg_drug = (sa_drug[...] + sb_drug[...]) * rcg
    xs0[...] = jnp.maximum(
        dot(agg_dis, w_dis[0]) + b_dis[0] + dot(xt_dis[...], w_dis[1]), 0.0)
    xs1[...] = jnp.maximum(
        dot(agg_dis, w_dis[2]) + b_dis[1] + dot(xt_dis[...], w_dis[3]), 0.0)
    xd0[...] = jnp.maximum(
        dot(agg_drug, w_drug[0]) + b_drug[0] + dot(xt_drug[...], w_drug[1]), 0.0)
    xd1[...] = jnp.maximum(
        dot(agg_drug, w_drug[2]) + b_drug[1] + dot(xt_drug[...], w_drug[3]), 0.0)


def _run_d0(sa_dis, sb_dis, cda, cdb, xt_dis, sa_drug, sb_drug, cga, cgb,
            xt_drug, w_dis, b_dis, w_drug, b_drug):
    r = 1000
    grid = (_N // r,)
    row = pl.BlockSpec((r, _H), lambda i: (i, 0))
    cblk = pl.BlockSpec((r, 1), lambda i: (i, 0))
    wblk = pl.BlockSpec((4, _H, _H), lambda i: (0, 0, 0))
    bblk = pl.BlockSpec((2, _H), lambda i: (0, 0))
    return pl.pallas_call(
        _d0_body,
        grid=grid,
        in_specs=[row, row, cblk, cblk, row, row, row, cblk, cblk, row,
                  wblk, bblk, wblk, bblk],
        out_specs=[row, row, row, row, row, row],
        out_shape=[jax.ShapeDtypeStruct((_N, _H), _f32)] * 6,
    )(sa_dis, sb_dis, cda, cdb, xt_dis, sa_drug, sb_drug, cga, cgb,
      xt_drug, w_dis, b_dis, w_drug, b_drug)


def _d1_body(gd0, gd1, gdx, gcd, gs0, gs1, gsx, gcs,
             w_d, b_d, w_s, b_s, wv_t, bv, wo_t, bo,
             w1, b1, w2, b2, w3r, out):
    dot = functools.partial(jnp.dot, preferred_element_type=_f32)
    demb = jnp.maximum(
        dot((gd0[...] + gd1[...]) * gcd[...], w_d[0])
        + b_d[0] + dot(gdx[...], w_d[1]), 0.0)
    semb = jnp.maximum(
        dot((gs0[...] + gs1[...]) * gcs[...], w_s[0])
        + b_s[0] + dot(gsx[...], w_s[1]), 0.0)
    datt = dot(dot(semb, wv_t[0]) + bv[0], wo_t[0]) + bo[0]
    satt = dot(dot(demb, wv_t[1]) + bv[1], wo_t[1]) + bo[1]
    li = jnp.concatenate([demb, semb, datt, satt], axis=-1)
    h = jnp.maximum(dot(li, w1[...]) + b1[0], 0.0)
    h = jnp.maximum(dot(h, w2[...]) + b2[0], 0.0)
    o = jnp.sum(h * w3r[0:1, :], axis=-1) + w3r[1, 0]
    out[...] = o.reshape(out.shape)


def _run_d1(gd0, gd1, gdx, gcd, gs0, gs1, gsx, gcs,
            w_d, b_d, w_s, b_s, wv_t, bv, wo_t, bo, w1, b1, w2, b2, w3r):
    r = 1024
    grid = (_B // r,)
    row = pl.BlockSpec((r, _H), lambda i: (i, 0))
    w2blk = pl.BlockSpec((2, _H, _H), lambda i: (0, 0, 0))
    bblk = pl.BlockSpec((2, _H), lambda i: (0, 0))
    full = lambda a: pl.BlockSpec(a.shape, lambda i: tuple(0 for _ in a.shape))
    return pl.pallas_call(
        _d1_body,
        grid=grid,
        in_specs=[row, row, row, row, row, row, row, row,
                  w2blk, bblk, w2blk, bblk, w2blk, bblk, w2blk, bblk,
                  full(w1), full(b1), full(w2), full(b2), full(w3r)],
        out_specs=[pl.BlockSpec((r // _H, _H), lambda i: (i, 0))],
        out_shape=[jax.ShapeDtypeStruct((_B // _H, _H), _f32)],
    )(gd0, gd1, gdx, gcd, gs0, gs1, gsx, gcs,
      w_d, b_d, w_s, b_s, wv_t, bv, wo_t, bo, w1, b1, w2, b2, w3r)[0]


def kernel(edge_index_dd, edge_index_rev, drug_idx, disease_idx, drug_table,
           disease_table, sage_wl, sage_bl, sage_wr, attn_in_w, attn_in_b,
           attn_out_w, attn_out_b, mlp_w1, mlp_b1, mlp_w2, mlp_b2, mlp_w3,
           mlp_b3):
    i32 = jnp.int32
    src_dd = edge_index_dd[0].astype(i32)
    dst_dd = edge_index_dd[1].astype(i32)
    src_rev = edge_index_rev[0].astype(i32)
    dst_rev = edge_index_rev[1].astype(i32)
    di = drug_idx.astype(i32)
    si = disease_idx.astype(i32)
    dip = di + _NPAD
    sip = si + _NPAD

    zrows = jnp.zeros((_RPT, _H), _f32)
    zc = jnp.zeros((_RPT,), _f32)
    ones_h = jnp.ones((_CH,), _f32)

    sum_dis0, cnt_dd = _segsum_cnt(drug_table, src_dd, dst_dd, zrows,
                                   ones_h, zc)
    sum_drug0, cnt_rv = _segsum_cnt(disease_table, src_rev, dst_rev, zrows,
                                    ones_h, zc)
    cda = cnt_dd[:_N].reshape(_N, 1)
    cdb = cnt_dd[_NPAD:_NPAD + _N].reshape(_N, 1)
    cga = cnt_rv[:_N].reshape(_N, 1)
    cgb = cnt_rv[_NPAD:_NPAD + _N].reshape(_N, 1)

    h = _H
    w_dis = jnp.stack([sage_wl[0, 0, 0], sage_wr[0, 0, 0],
                       sage_wl[1, 0, 0], sage_wr[1, 0, 0]])
    b_dis = jnp.stack([sage_bl[0, 0, 0], sage_bl[1, 0, 0]])
    w_drug = jnp.stack([sage_wl[0, 0, 1], sage_wr[0, 0, 1],
                        sage_wl[1, 0, 1], sage_wr[1, 0, 1]])
    b_drug = jnp.stack([sage_bl[0, 0, 1], sage_bl[1, 0, 1]])

    xs0, xs1, xd0, xd1, rc_dis, rc_drug = _run_d0(
        sum_dis0[:_N], sum_dis0[_NPAD:_NPAD + _N], cda, cdb, disease_table,
        sum_drug0[:_N], sum_drug0[_NPAD:_NPAD + _N], cga, cgb, drug_table,
        w_dis, b_dis, w_drug, b_drug)

    sum_drug1 = _segsum(xs0, src_rev, dst_rev, zrows)
    sum_dis1 = _segsum(xd1, src_dd, dst_dd, zrows)

    gd0, gd1, gdx, gcd, gs0, gs1, gsx, gcs = _gather(
        sum_drug1, xd0, rc_drug, sum_dis1, xs1, rc_dis, di, dip, si, sip)

    w_d = jnp.stack([sage_wl[0, 1, 1], sage_wr[0, 1, 1]])
    b_d = jnp.stack([sage_bl[0, 1, 1], sage_bl[0, 1, 1]])
    w_s = jnp.stack([sage_wl[1, 1, 0], sage_wr[1, 1, 0]])
    b_s = jnp.stack([sage_bl[1, 1, 0], sage_bl[1, 1, 0]])
    wv_t = jnp.stack([attn_in_w[0, 2 * h:3 * h].T, attn_in_w[1, 2 * h:3 * h].T])
    bv = jnp.stack([attn_in_b[0, 2 * h:3 * h], attn_in_b[1, 2 * h:3 * h]])
    wo_t = jnp.stack([attn_out_w[0].T, attn_out_w[1].T])
    bo = jnp.stack([attn_out_b[0], attn_out_b[1]])
    b1 = mlp_b1.reshape(1, -1)
    b2 = mlp_b2.reshape(1, -1)
    w3r = jnp.concatenate(
        [mlp_w3[:, 0:1].T, jnp.full((1, _H), mlp_b3[0], _f32)], axis=0)

    out = _run_d1(gd0, gd1, gdx, gcd, gs0, gs1, gsx, gcs,
                  w_d, b_d, w_s, b_s, wv_t, bv, wo_t, bo,
                  mlp_w1, b1, mlp_w2, b2, w3r)
    return out.reshape(_B)
